# baseline (speedup 1.0000x reference)
import jax
import jax.numpy as jnp
from jax.experimental import pallas as pl
from jax.experimental.pallas import tpu as pltpu

EPS = 0.1
TB = 4


def _select_body(dpad_ref, u_ref, l_ref):
    scores = jnp.where(dpad_ref[...] != 1, u_ref[...], -1.0)
    m = jnp.max(scores, axis=1, keepdims=True)
    col = jax.lax.broadcasted_iota(jnp.int32, scores.shape, 1)
    l = jnp.min(jnp.where(scores == m, col, 2**30), axis=1, keepdims=True)
    l_ref[...] = jnp.broadcast_to(l, l_ref.shape)


def _stream_body(l_sp, emb_ref, out_ref):
    i = pl.program_id(0)
    out_ref[...] = emb_ref[...]
    for t in range(TB):
        lt = l_sp[i * TB + t]
        row = emb_ref[t, pl.ds(lt, 1), :]
        norm = jnp.sqrt(jnp.sum(row * row, axis=1, keepdims=True))
        out_ref[t, pl.ds(lt, 1), :] = row + EPS * row / norm


def kernel(emb, data, dpadder, emb_matr):
    tlen, bz, d = emb.shape
    u = jax.random.uniform(jax.random.key(42), (tlen, bz))

    l_wide = pl.pallas_call(
        _select_body,
        out_shape=jax.ShapeDtypeStruct((tlen, 128), jnp.int32),
    )(dpadder, u)
    l = l_wide[:, 0]

    a = pl.pallas_call(
        _stream_body,
        grid_spec=pltpu.PrefetchScalarGridSpec(
            num_scalar_prefetch=1,
            grid=(tlen // TB,),
            in_specs=[pl.BlockSpec((TB, bz, d), lambda i, l_sp: (i, 0, 0))],
            out_specs=pl.BlockSpec((TB, bz, d), lambda i, l_sp: (i, 0, 0)),
        ),
        out_shape=jax.ShapeDtypeStruct((tlen, bz, d), emb.dtype),
    )(l, emb)
    return a, l


# speedup vs baseline: 1.1078x; 1.1078x over previous
import jax
import jax.numpy as jnp
from jax.experimental import pallas as pl
from jax.experimental.pallas import tpu as pltpu

EPS = 0.1
TB = 20


def _select_body(dpad_ref, u_ref, l_ref):
    scores = jnp.where(dpad_ref[...] != 1, u_ref[...], -1.0)
    m = jnp.max(scores, axis=1, keepdims=True)
    col = jax.lax.broadcasted_iota(jnp.int32, scores.shape, 1)
    l = jnp.min(jnp.where(scores == m, col, 2**30), axis=1, keepdims=True)
    l_ref[...] = jnp.broadcast_to(l, l_ref.shape)


def _stream_body(l_sp, emb_ref, out_ref):
    i = pl.program_id(0)
    out_ref[...] = emb_ref[...]
    for t in range(TB):
        lt = l_sp[i * TB + t]
        row = emb_ref[t, pl.ds(lt, 1), :]
        norm = jnp.sqrt(jnp.sum(row * row, axis=1, keepdims=True))
        out_ref[t, pl.ds(lt, 1), :] = row + EPS * row / norm


def kernel(emb, data, dpadder, emb_matr):
    tlen, bz, d = emb.shape
    u = jax.random.uniform(jax.random.key(42), (tlen, bz))

    l_wide = pl.pallas_call(
        _select_body,
        out_shape=jax.ShapeDtypeStruct((tlen, 128), jnp.int32),
    )(dpadder, u)
    l = l_wide[:, 0]

    a = pl.pallas_call(
        _stream_body,
        grid_spec=pltpu.PrefetchScalarGridSpec(
            num_scalar_prefetch=1,
            grid=(tlen // TB,),
            in_specs=[pl.BlockSpec((TB, bz, d), lambda i, l_sp: (i, 0, 0))],
            out_specs=pl.BlockSpec((TB, bz, d), lambda i, l_sp: (i, 0, 0)),
        ),
        out_shape=jax.ShapeDtypeStruct((tlen, bz, d), emb.dtype),
    )(l, emb)
    return a, l


# speedup vs baseline: 1.1090x; 1.0011x over previous
import jax
import jax.numpy as jnp
from jax.experimental import pallas as pl
from jax.experimental.pallas import tpu as pltpu

EPS = 0.1
TB = 25


def _select_body(dpad_ref, u_ref, l_ref):
    scores = jnp.where(dpad_ref[...] != 1, u_ref[...], -1.0)
    m = jnp.max(scores, axis=1, keepdims=True)
    col = jax.lax.broadcasted_iota(jnp.int32, scores.shape, 1)
    l = jnp.min(jnp.where(scores == m, col, 2**30), axis=1, keepdims=True)
    l_ref[...] = jnp.broadcast_to(l, l_ref.shape)


def _stream_body(l_sp, emb_ref, out_ref):
    i = pl.program_id(0)
    out_ref[...] = emb_ref[...]
    for t in range(TB):
        lt = l_sp[i * TB + t]
        row = emb_ref[t, pl.ds(lt, 1), :]
        norm = jnp.sqrt(jnp.sum(row * row, axis=1, keepdims=True))
        out_ref[t, pl.ds(lt, 1), :] = row + EPS * row / norm


def kernel(emb, data, dpadder, emb_matr):
    tlen, bz, d = emb.shape
    u = jax.random.uniform(jax.random.key(42), (tlen, bz))

    l_wide = pl.pallas_call(
        _select_body,
        out_shape=jax.ShapeDtypeStruct((tlen, 128), jnp.int32),
    )(dpadder, u)
    l = l_wide[:, 0]

    a = pl.pallas_call(
        _stream_body,
        grid_spec=pltpu.PrefetchScalarGridSpec(
            num_scalar_prefetch=1,
            grid=(tlen // TB,),
            in_specs=[pl.BlockSpec((TB, bz, d), lambda i, l_sp: (i, 0, 0))],
            out_specs=pl.BlockSpec((TB, bz, d), lambda i, l_sp: (i, 0, 0)),
        ),
        out_shape=jax.ShapeDtypeStruct((tlen, bz, d), emb.dtype),
    )(l, emb)
    return a, l
